# baseline (speedup 1.0000x reference)
import functools
import math

import jax
import jax.numpy as jnp
from jax import lax
from jax.experimental import pallas as pl
from jax.experimental.pallas import tpu as pltpu
from jax.experimental.pallas import tpu_sc as plsc

D_MODEL = 512
SCALE = float(math.sqrt(D_MODEL))
LANES = 16


def _build_sc_kernel(n_rows: int):
    info = plsc.get_sparse_core_info()
    nw = info.num_cores * info.num_subcores
    rows_per_w = n_rows // nw
    chunk = 128
    n_chunks = rows_per_w // chunk

    mesh = plsc.VectorSubcoreMesh(core_axis_name="c", subcore_axis_name="s")

    @functools.partial(
        pl.kernel,
        mesh=mesh,
        out_type=jax.ShapeDtypeStruct((n_rows, D_MODEL), jnp.float32),
        scratch_types=[
            pltpu.VMEM((rows_per_w,), jnp.int32),
            pltpu.VMEM((chunk, D_MODEL), jnp.float32),
            pltpu.SemaphoreType.DMA,
        ],
    )
    def sc_kernel(idx_hbm, table_hbm, out_hbm, idx_v, rows_v, sem):
        wid = lax.axis_index("s") * info.num_cores + lax.axis_index("c")
        base = wid * rows_per_w
        pltpu.sync_copy(idx_hbm.at[pl.ds(base, rows_per_w)], idx_v)

        def chunk_body(i, _):
            pltpu.async_copy(
                table_hbm.at[idx_v.at[pl.ds(i * chunk, chunk)]], rows_v, sem
            ).wait()

            def scale_row(r, _):
                for j in range(D_MODEL // LANES):
                    sl = pl.ds(j * LANES, LANES)
                    rows_v[r, sl] = rows_v[r, sl] * SCALE
                return 0

            lax.fori_loop(0, chunk, scale_row, 0)
            pltpu.sync_copy(rows_v, out_hbm.at[pl.ds(base + i * chunk, chunk)])
            return 0

        lax.fori_loop(0, n_chunks, chunk_body, 0)

    return sc_kernel


def kernel(inputs, table):
    b, s = inputs.shape
    n_rows = b * s
    idx_flat = inputs.reshape(n_rows).astype(jnp.int32)
    out = _build_sc_kernel(n_rows)(idx_flat, table)
    return out.reshape(b, s, D_MODEL)


# speedup vs baseline: 1.1364x; 1.1364x over previous
import functools
import math

import jax
import jax.numpy as jnp
from jax import lax
from jax.experimental import pallas as pl
from jax.experimental.pallas import tpu as pltpu
from jax.experimental.pallas import tpu_sc as plsc

D_MODEL = 512
SCALE = float(math.sqrt(D_MODEL))
LANES = 16
NBUF = 4
CHUNK = 40


def _build_sc_kernel(n_rows: int):
    info = plsc.get_sparse_core_info()
    nw = info.num_cores * info.num_subcores
    rows_per_w = n_rows // nw
    n_chunks = rows_per_w // CHUNK
    n_outer = n_chunks // NBUF

    mesh = plsc.VectorSubcoreMesh(core_axis_name="c", subcore_axis_name="s")

    @functools.partial(
        pl.kernel,
        mesh=mesh,
        out_type=jax.ShapeDtypeStruct((n_rows, D_MODEL), jnp.float32),
        scratch_types=(
            [pltpu.VMEM((rows_per_w,), jnp.int32)]
            + [pltpu.VMEM((CHUNK, D_MODEL), jnp.float32) for _ in range(NBUF)]
            + [pltpu.SemaphoreType.DMA for _ in range(2 * NBUF)]
        ),
    )
    def sc_kernel(idx_hbm, table_hbm, out_hbm, idx_v, *bufs_and_sems):
        bufs = bufs_and_sems[:NBUF]
        gsem = bufs_and_sems[NBUF:2 * NBUF]
        ssem = bufs_and_sems[2 * NBUF:]

        wid = lax.axis_index("s") * info.num_cores + lax.axis_index("c")
        base = wid * rows_per_w
        pltpu.sync_copy(idx_hbm.at[pl.ds(base, rows_per_w)], idx_v)

        def gather(chunk_id, b):
            pltpu.make_async_copy(
                table_hbm.at[idx_v.at[pl.ds(chunk_id * CHUNK, CHUNK)]],
                bufs[b], gsem[b],
            ).start()

        def gather_wait(chunk_id, b):
            pltpu.make_async_copy(
                table_hbm.at[idx_v.at[pl.ds(chunk_id * CHUNK, CHUNK)]],
                bufs[b], gsem[b],
            ).wait()

        def scatter(chunk_id, b):
            pltpu.make_async_copy(
                bufs[b], out_hbm.at[pl.ds(base + chunk_id * CHUNK, CHUNK)],
                ssem[b],
            ).start()

        def scatter_wait(chunk_id, b):
            pltpu.make_async_copy(
                bufs[b], out_hbm.at[pl.ds(base + chunk_id * CHUNK, CHUNK)],
                ssem[b],
            ).wait()

        for b in range(NBUF - 1):
            gather(b, b)

        def outer_body(o, _):
            for b in range(NBUF):
                g = o * NBUF + b
                gather_wait(g, b)

                def scale_row(r, _):
                    for j in range(D_MODEL // LANES):
                        sl = pl.ds(j * LANES, LANES)
                        bufs[b][r, sl] = bufs[b][r, sl] * SCALE
                    return 0

                lax.fori_loop(0, CHUNK, scale_row, 0)
                scatter(g, b)
                pb = (b - 1) % NBUF
                if b == 0:
                    @pl.when(o > 0)
                    def _():
                        scatter_wait(g - 1, pb)
                        gather(g + NBUF - 1, pb)
                    @pl.when(o == 0)
                    def _():
                        gather(g + NBUF - 1, pb)
                else:
                    scatter_wait(g - 1, pb)

                    @pl.when(o < n_outer - 1)
                    def _():
                        gather(g + NBUF - 1, pb)
            return 0

        lax.fori_loop(0, n_outer, outer_body, 0)
        scatter_wait(n_chunks - 1, NBUF - 1)

    return sc_kernel


def kernel(inputs, table):
    b, s = inputs.shape
    n_rows = b * s
    idx_flat = inputs.reshape(n_rows).astype(jnp.int32)
    out = _build_sc_kernel(n_rows)(idx_flat, table)
    return out.reshape(b, s, D_MODEL)


# speedup vs baseline: 3.5848x; 3.1546x over previous
import functools
import math

import jax
import jax.numpy as jnp
from jax import lax
from jax.experimental import pallas as pl
from jax.experimental.pallas import tpu as pltpu
from jax.experimental.pallas import tpu_sc as plsc

D_MODEL = 512
SCALE = float(math.sqrt(D_MODEL))
LANES = 16
NBUF = 4
CHUNK = 40


def _build_sc_kernel(n_rows: int):
    info = plsc.get_sparse_core_info()
    nw = info.num_cores * info.num_subcores
    rows_per_w = n_rows // nw
    n_chunks = rows_per_w // CHUNK
    n_outer = n_chunks // NBUF

    mesh = plsc.VectorSubcoreMesh(core_axis_name="c", subcore_axis_name="s")

    @functools.partial(
        pl.kernel,
        mesh=mesh,
        out_type=jax.ShapeDtypeStruct((n_rows, D_MODEL), jnp.float32),
        scratch_types=(
            [pltpu.VMEM((rows_per_w,), jnp.int32)]
            + [pltpu.VMEM((CHUNK, D_MODEL), jnp.float32) for _ in range(NBUF)]
            + [pltpu.SemaphoreType.DMA for _ in range(2 * NBUF)]
        ),
    )
    def sc_kernel(idx_hbm, table_hbm, out_hbm, idx_v, *bufs_and_sems):
        bufs = bufs_and_sems[:NBUF]
        gsem = bufs_and_sems[NBUF:2 * NBUF]
        ssem = bufs_and_sems[2 * NBUF:]

        wid = lax.axis_index("s") * info.num_cores + lax.axis_index("c")
        base = wid * rows_per_w
        pltpu.sync_copy(idx_hbm.at[pl.ds(base, rows_per_w)], idx_v)

        def gather(chunk_id, b):
            pltpu.make_async_copy(
                table_hbm.at[idx_v.at[pl.ds(chunk_id * CHUNK, CHUNK)]],
                bufs[b], gsem[b],
            ).start()

        def gather_wait(chunk_id, b):
            pltpu.make_async_copy(
                table_hbm.at[idx_v.at[pl.ds(chunk_id * CHUNK, CHUNK)]],
                bufs[b], gsem[b],
            ).wait()

        def scatter(chunk_id, b):
            pltpu.make_async_copy(
                bufs[b], out_hbm.at[pl.ds(base + chunk_id * CHUNK, CHUNK)],
                ssem[b],
            ).start()

        def scatter_wait(chunk_id, b):
            pltpu.make_async_copy(
                bufs[b], out_hbm.at[pl.ds(base + chunk_id * CHUNK, CHUNK)],
                ssem[b],
            ).wait()

        for b in range(NBUF - 1):
            gather(b, b)

        def outer_body(o, _):
            for b in range(NBUF):
                g = o * NBUF + b
                gather_wait(g, b)

                def scale_row(r, _):
                    for j in range(D_MODEL // LANES):
                        sl = pl.ds(j * LANES, LANES)
                        bufs[b][r, sl] = bufs[b][r, sl] * SCALE
                    return 0

                lax.fori_loop(0, CHUNK, scale_row, 0)
                scatter(g, b)
                pb = (b - 1) % NBUF
                if b == 0:
                    @pl.when(o > 0)
                    def _():
                        scatter_wait(g - 1, pb)
                        gather(g + NBUF - 1, pb)
                    @pl.when(o == 0)
                    def _():
                        gather(g + NBUF - 1, pb)
                else:
                    scatter_wait(g - 1, pb)

                    @pl.when(o < n_outer - 1)
                    def _():
                        gather(g + NBUF - 1, pb)
            return 0

        lax.fori_loop(0, n_outer, outer_body, 0)
        scatter_wait(n_chunks - 1, NBUF - 1)

    return sc_kernel


def kernel(inputs, table):
    b, s = inputs.shape
    n_rows = b * s
    idx_flat = jnp.transpose(inputs).reshape(n_rows).astype(jnp.int32)
    out = _build_sc_kernel(n_rows)(idx_flat, table)
    return out.reshape(s, b, D_MODEL).transpose(1, 0, 2)
